# Initial kernel scaffold; baseline (speedup 1.0000x reference)
import jax
import jax.numpy as jnp
from jax.experimental import pallas as pl


def kernel(values, channel_ids, time_ids, proj_w, proj_b, channel_table, time_table):
    raise NotImplementedError("write your pallas kernel here")


# speedup vs baseline: 2.8630x; 2.8630x over previous
import functools

import jax
import jax.numpy as jnp
from jax import lax
from jax.experimental import pallas as pl
from jax.experimental.pallas import tpu as pltpu
from jax.experimental.pallas import tpu_sc as plsc

B, L, D = 4096, 200, 128
N_CH, N_T = 256, 200
N = B * L
NC, NS = 2, 16
NW = NC * NS
TOK_PER_W = N // NW
C = 256
CHUNKS = TOK_PER_W // C


def _sc_embed(vals_hbm, cid_hbm, tid_hbm, ch_hbm, t_hbm, w_hbm, b_hbm,
              out_hbm, ch_v, t_v, w_v, b_v, vals_v, cid_v, tid_v, out_v):
    wid = lax.axis_index("s") * NC + lax.axis_index("c")
    base = wid * TOK_PER_W

    pltpu.sync_copy(ch_hbm, ch_v)
    pltpu.sync_copy(t_hbm, t_v)
    pltpu.sync_copy(w_hbm, w_v)
    pltpu.sync_copy(b_hbm, b_v)

    iota = lax.iota(jnp.int32, 16)
    wregs = [w_v[pl.ds(16 * k, 16)] for k in range(8)]
    bregs = [b_v[pl.ds(16 * k, 16)] for k in range(8)]
    offs = [iota + (16 * k) for k in range(8)]

    def chunk_body(ci, carry):
        tok0 = base + ci * C
        pltpu.sync_copy(vals_hbm.at[pl.ds(tok0, C)], vals_v)
        pltpu.sync_copy(cid_hbm.at[pl.ds(tok0, C)], cid_v)
        pltpu.sync_copy(tid_hbm.at[pl.ds(tok0, C)], tid_v)

        def tok_body(j, c2):
            jsplat = jnp.full((16,), j, jnp.int32)
            cidx = plsc.load_gather(cid_v, [jsplat]) * 128
            tidx = plsc.load_gather(tid_v, [jsplat]) * 128
            val = plsc.load_gather(vals_v, [jsplat])
            for k in range(8):
                chv = plsc.load_gather(ch_v, [cidx + offs[k]])
                ttv = plsc.load_gather(t_v, [tidx + offs[k]])
                out_v[j, pl.ds(16 * k, 16)] = (chv + ttv + bregs[k]) + val * wregs[k]
            return c2

        lax.fori_loop(0, C, tok_body, 0, unroll=2)
        pltpu.sync_copy(out_v, out_hbm.at[pl.ds(tok0, C)])
        return carry

    lax.fori_loop(0, CHUNKS, chunk_body, 0)


def kernel(values, channel_ids, time_ids, proj_w, proj_b, channel_table, time_table):
    vals = values.reshape(N)
    cid = channel_ids.astype(jnp.int32).reshape(N)
    tid = time_ids.astype(jnp.int32).reshape(N)
    ch_flat = channel_table.reshape(N_CH * D)
    t_flat = time_table.reshape(N_T * D)
    w = proj_w.reshape(D)

    mesh = plsc.VectorSubcoreMesh(core_axis_name="c", subcore_axis_name="s")
    f = functools.partial(
        pl.kernel,
        mesh=mesh,
        out_type=jax.ShapeDtypeStruct((N, D), jnp.float32),
        compiler_params=pltpu.CompilerParams(needs_layout_passes=False),
        scratch_types=[
            pltpu.VMEM((N_CH * D,), jnp.float32),
            pltpu.VMEM((N_T * D,), jnp.float32),
            pltpu.VMEM((D,), jnp.float32),
            pltpu.VMEM((D,), jnp.float32),
            pltpu.VMEM((C,), jnp.float32),
            pltpu.VMEM((C,), jnp.int32),
            pltpu.VMEM((C,), jnp.int32),
            pltpu.VMEM((C, D), jnp.float32),
        ],
    )(_sc_embed)
    out = f(vals, cid, tid, ch_flat, t_flat, w, proj_b)
    return out.reshape(B, L, D)


# speedup vs baseline: 3.5019x; 1.2232x over previous
import functools

import jax
import jax.numpy as jnp
from jax import lax
from jax.experimental import pallas as pl
from jax.experimental.pallas import tpu as pltpu
from jax.experimental.pallas import tpu_sc as plsc

B, L, D = 4096, 200, 128
N_CH, N_T = 256, 200
N = B * L
NC, NS = 2, 16
NW = NC * NS
TOK_PER_W = N // NW
C = 256
CHUNKS = TOK_PER_W // C
G = 16


def _sc_embed(vals_hbm, cid_hbm, tid_hbm, ch_hbm, t_hbm, w_hbm, b_hbm,
              out_hbm, ch_v, t_v, w_v, b_v, vals0, vals1, cid0, cid1,
              tid0, tid1, out_v, in_sem, out_sem):
    vals_b = (vals0, vals1)
    cid_b = (cid0, cid1)
    tid_b = (tid0, tid1)
    wid = lax.axis_index("s") * NC + lax.axis_index("c")
    base = wid * TOK_PER_W

    pltpu.sync_copy(ch_hbm, ch_v)
    pltpu.sync_copy(t_hbm, t_v)
    pltpu.sync_copy(w_hbm, w_v)
    pltpu.sync_copy(b_hbm, b_v)

    iota = lax.iota(jnp.int32, 16)
    wregs = [w_v[pl.ds(16 * k, 16)] for k in range(8)]
    bregs = [b_v[pl.ds(16 * k, 16)] for k in range(8)]
    offs = [iota + (16 * k) for k in range(8)]

    def start_in(ci, b):
        tok0 = base + ci * C
        pltpu.async_copy(vals_hbm.at[pl.ds(tok0, C)], vals_b[b], in_sem.at[b])
        pltpu.async_copy(cid_hbm.at[pl.ds(tok0, C)], cid_b[b], in_sem.at[b])
        pltpu.async_copy(tid_hbm.at[pl.ds(tok0, C)], tid_b[b], in_sem.at[b])

    def wait_in(b):
        pltpu.make_async_copy(vals_hbm.at[pl.ds(0, C)], vals_b[b], in_sem.at[b]).wait()
        pltpu.make_async_copy(cid_hbm.at[pl.ds(0, C)], cid_b[b], in_sem.at[b]).wait()
        pltpu.make_async_copy(tid_hbm.at[pl.ds(0, C)], tid_b[b], in_sem.at[b]).wait()

    def start_out(ci, b):
        tok0 = base + ci * C
        pltpu.async_copy(out_v.at[b], out_hbm.at[pl.ds(tok0, C)], out_sem.at[b])

    def wait_out(b):
        pltpu.make_async_copy(out_v.at[b], out_hbm.at[pl.ds(0, C)], out_sem.at[b]).wait()

    def compute(b):
        def group(g, carry):
            j0 = g * G
            jsplat = jnp.full((16,), j0, jnp.int32)
            for j in range(G):
                js = jsplat + j
                cidx = plsc.load_gather(cid_b[b], [js]) * 128
                tidx = plsc.load_gather(tid_b[b], [js]) * 128
                val = plsc.load_gather(vals_b[b], [js])
                for k in range(8):
                    chv = plsc.load_gather(ch_v, [cidx + offs[k]])
                    ttv = plsc.load_gather(t_v, [tidx + offs[k]])
                    out_v[b, j0 + j, pl.ds(16 * k, 16)] = (chv + ttv + bregs[k]) + val * wregs[k]
            return carry
        lax.fori_loop(0, C // G, group, 0)

    start_in(0, 0)
    start_in(1, 1)

    def pair(p, carry):
        for b in range(2):
            ci = p * 2 + b
            wait_in(b)

            @pl.when(ci >= 2)
            def _():
                wait_out(b)

            compute(b)
            start_out(ci, b)

            @pl.when(ci + 2 < CHUNKS)
            def _():
                start_in(ci + 2, b)
        return carry

    lax.fori_loop(0, CHUNKS // 2, pair, 0)
    wait_out(0)
    wait_out(1)


def kernel(values, channel_ids, time_ids, proj_w, proj_b, channel_table, time_table):
    vals = values.reshape(N)
    cid = channel_ids.astype(jnp.int32).reshape(N)
    tid = time_ids.astype(jnp.int32).reshape(N)
    ch_flat = channel_table.reshape(N_CH * D)
    t_flat = time_table.reshape(N_T * D)
    w = proj_w.reshape(D)

    mesh = plsc.VectorSubcoreMesh(core_axis_name="c", subcore_axis_name="s")
    f = functools.partial(
        pl.kernel,
        mesh=mesh,
        out_type=jax.ShapeDtypeStruct((N, D), jnp.float32),
        compiler_params=pltpu.CompilerParams(needs_layout_passes=False),
        scratch_types=[
            pltpu.VMEM((N_CH * D,), jnp.float32),
            pltpu.VMEM((N_T * D,), jnp.float32),
            pltpu.VMEM((D,), jnp.float32),
            pltpu.VMEM((D,), jnp.float32),
            pltpu.VMEM((C,), jnp.float32),
            pltpu.VMEM((C,), jnp.float32),
            pltpu.VMEM((C,), jnp.int32),
            pltpu.VMEM((C,), jnp.int32),
            pltpu.VMEM((C,), jnp.int32),
            pltpu.VMEM((C,), jnp.int32),
            pltpu.VMEM((2, C, D), jnp.float32),
            pltpu.SemaphoreType.DMA((2,)),
            pltpu.SemaphoreType.DMA((2,)),
        ],
    )(_sc_embed)
    out = f(vals, cid, tid, ch_flat, t_flat, w, proj_b)
    return out.reshape(B, L, D)


# speedup vs baseline: 12.8230x; 3.6618x over previous
import functools

import jax
import jax.numpy as jnp
from jax import lax
from jax.experimental import pallas as pl
from jax.experimental.pallas import tpu as pltpu
from jax.experimental.pallas import tpu_sc as plsc

B, L, D = 4096, 200, 128
N_CH, N_T = 256, 200
N = B * L
NC, NS = 2, 16
NW = NC * NS
TOK_PER_W = N // NW
C = 256
CHUNKS = TOK_PER_W // C
G = 8


def _sc_embed(vals_hbm, cid_hbm, tid_hbm, ch_hbm, t_hbm, w_hbm, b_hbm,
              out_hbm, ch_v, t_v, w_v, b_v, vals0, vals1, cid0, cid1,
              tid0, tid1, out_v, in_sem, out_sem):
    vals_b = (vals0, vals1)
    cid_b = (cid0, cid1)
    tid_b = (tid0, tid1)
    wid = lax.axis_index("s") * NC + lax.axis_index("c")
    base = wid * TOK_PER_W

    pltpu.sync_copy(ch_hbm, ch_v)
    pltpu.sync_copy(t_hbm, t_v)
    pltpu.sync_copy(w_hbm, w_v)
    pltpu.sync_copy(b_hbm, b_v)

    iota = lax.iota(jnp.int32, 16)
    wregs = [w_v[pl.ds(16 * k, 16)] for k in range(8)]
    bregs = [b_v[pl.ds(16 * k, 16)] for k in range(8)]
    offs = [iota + (16 * k) for k in range(8)]

    def fold(g, carry):
        for k in range(8):
            sl = pl.ds(g * 128 + 16 * k, 16)
            ch_v[sl] = ch_v[sl] + bregs[k]
        return carry

    lax.fori_loop(0, N_CH, fold, 0)

    def start_in(ci, b):
        tok0 = base + ci * C
        pltpu.async_copy(vals_hbm.at[pl.ds(tok0, C)], vals_b[b], in_sem.at[b])
        pltpu.async_copy(cid_hbm.at[pl.ds(tok0, C)], cid_b[b], in_sem.at[b])
        pltpu.async_copy(tid_hbm.at[pl.ds(tok0, C)], tid_b[b], in_sem.at[b])

    def wait_in(b):
        pltpu.make_async_copy(vals_hbm.at[pl.ds(0, C)], vals_b[b], in_sem.at[b]).wait()
        pltpu.make_async_copy(cid_hbm.at[pl.ds(0, C)], cid_b[b], in_sem.at[b]).wait()
        pltpu.make_async_copy(tid_hbm.at[pl.ds(0, C)], tid_b[b], in_sem.at[b]).wait()

    def start_out(ci, b):
        tok0 = base + ci * C
        pltpu.async_copy(out_v.at[b], out_hbm.at[pl.ds(tok0, C)], out_sem.at[b])

    def wait_out(b):
        pltpu.make_async_copy(out_v.at[b], out_hbm.at[pl.ds(0, C)], out_sem.at[b]).wait()

    def compute(b):
        cid_r, tid_r, val_r = cid_b[b], tid_b[b], vals_b[b]

        def group(g, carry):
            j0 = g * G
            jsplat = jnp.full((16,), j0, jnp.int32)
            cidx, tidx, val = [], [], []
            for j in range(G):
                js = jsplat + j
                cidx.append(plsc.load_gather(cid_r, [js]) * 128)
                tidx.append(plsc.load_gather(tid_r, [js]) * 128)
                val.append(plsc.load_gather(val_r, [js]))
            for j in range(G):
                chs = [plsc.load_gather(ch_v, [cidx[j] + offs[k]]) for k in range(8)]
                tts = [plsc.load_gather(t_v, [tidx[j] + offs[k]]) for k in range(8)]
                for k in range(8):
                    out_v[b, j0 + j, pl.ds(16 * k, 16)] = (chs[k] + tts[k]) + val[j] * wregs[k]
            return carry

        lax.fori_loop(0, C // G, group, 0)

    start_in(0, 0)
    start_in(1, 1)

    def pair(p, carry):
        for b in range(2):
            ci = p * 2 + b
            wait_in(b)

            @pl.when(ci >= 2)
            def _():
                wait_out(b)

            compute(b)
            start_out(ci, b)

            @pl.when(ci + 2 < CHUNKS)
            def _():
                start_in(ci + 2, b)
        return carry

    lax.fori_loop(0, CHUNKS // 2, pair, 0)
    wait_out(0)
    wait_out(1)


def kernel(values, channel_ids, time_ids, proj_w, proj_b, channel_table, time_table):
    vals = values.reshape(N)
    cid = channel_ids.astype(jnp.int32).reshape(N)
    tid = time_ids.astype(jnp.int32).reshape(N)
    ch_flat = channel_table.reshape(N_CH * D)
    t_flat = time_table.reshape(N_T * D)
    w = proj_w.reshape(D)

    mesh = plsc.VectorSubcoreMesh(core_axis_name="c", subcore_axis_name="s")
    f = functools.partial(
        pl.kernel,
        mesh=mesh,
        out_type=jax.ShapeDtypeStruct((N, D), jnp.float32),
        compiler_params=pltpu.CompilerParams(needs_layout_passes=False),
        scratch_types=[
            pltpu.VMEM((N_CH * D,), jnp.float32),
            pltpu.VMEM((N_T * D,), jnp.float32),
            pltpu.VMEM((D,), jnp.float32),
            pltpu.VMEM((D,), jnp.float32),
            pltpu.VMEM((C,), jnp.float32),
            pltpu.VMEM((C,), jnp.float32),
            pltpu.VMEM((C,), jnp.int32),
            pltpu.VMEM((C,), jnp.int32),
            pltpu.VMEM((C,), jnp.int32),
            pltpu.VMEM((C,), jnp.int32),
            pltpu.VMEM((2, C, D), jnp.float32),
            pltpu.SemaphoreType.DMA((2,)),
            pltpu.SemaphoreType.DMA((2,)),
        ],
    )(_sc_embed)
    out = f(vals, cid, tid, ch_flat, t_flat, w, proj_b)
    return out.reshape(B, L, D)
